# Initial kernel scaffold; baseline (speedup 1.0000x reference)
import jax
import jax.numpy as jnp
from jax.experimental import pallas as pl


def kernel(batch_mention_emb, mention_type_ids, table, gamma, beta):
    raise NotImplementedError("write your pallas kernel here")


# speedup vs baseline: 2.3596x; 2.3596x over previous
import functools

import jax
import jax.numpy as jnp
import numpy as np
from jax import lax
from jax.experimental import pallas as pl
from jax.experimental.pallas import tpu as pltpu
from jax.experimental.pallas import tpu_sc as plsc

H = 128
LANES = 16
VPR = H // LANES
T = 128
NBUF = 2
NCORES = 2
NSUB = 16
NW = NCORES * NSUB
EPS = 1e-5


def _allreduce_sum(v):
    lanes = lax.iota(jnp.int32, LANES)
    for sh in (1, 2, 4, 8):
        p = jnp.bitwise_xor(lanes, sh)
        v = v + v.at[p].get(mode="promise_in_bounds", unique_indices=True)
    return v


def _rsqrt(v):
    vi = lax.bitcast_convert_type(v, jnp.int32)
    seed = jnp.full((LANES,), 0x5F3759DF, dtype=jnp.int32) - jnp.right_shift(vi, 1)
    y = lax.bitcast_convert_type(seed, jnp.float32)
    half = v * 0.5
    for _ in range(3):
        y = y * (1.5 - half * y * y)
    return y


def _ln_chunk(emb_r, rows_r, out_r, gam_r, bet_r):
    gams = [gam_r[pl.ds(j * LANES, LANES)] for j in range(VPR)]
    bets = [bet_r[pl.ds(j * LANES, LANES)] for j in range(VPR)]

    @pl.loop(0, T)
    def _token(t):
        x = [emb_r[t, pl.ds(j * LANES, LANES)] + rows_r[t, pl.ds(j * LANES, LANES)]
             for j in range(VPR)]
        s = x[0]
        sq = x[0] * x[0]
        for j in range(1, VPR):
            s = s + x[j]
            sq = sq + x[j] * x[j]
        m_v = _allreduce_sum(s) * (1.0 / H)
        var = _allreduce_sum(sq) * (1.0 / H) - m_v * m_v + EPS
        r_v = _rsqrt(var)
        for j in range(VPR):
            out_r[t, pl.ds(j * LANES, LANES)] = (x[j] - m_v) * r_v * gams[j] + bets[j]


def _make_sc_call(n_tokens):
    tok_per_w = n_tokens // NW
    nchunk = tok_per_w // T
    mesh = plsc.VectorSubcoreMesh(core_axis_name="c", subcore_axis_name="s",
                                  num_cores=NCORES, num_subcores=NSUB)

    @functools.partial(
        pl.kernel,
        out_type=jax.ShapeDtypeStruct((n_tokens, H), jnp.float32),
        mesh=mesh,
        scratch_types=[
            pltpu.VMEM((nchunk, T), jnp.int32),
            pltpu.VMEM((H,), jnp.float32),
            pltpu.VMEM((H,), jnp.float32),
            pltpu.VMEM((NBUF, T, H), jnp.float32),
            pltpu.VMEM((NBUF, T, H), jnp.float32),
            pltpu.VMEM((NBUF, T, H), jnp.float32),
            pltpu.SemaphoreType.DMA,
            pltpu.SemaphoreType.DMA,
            pltpu.SemaphoreType.DMA,
            pltpu.SemaphoreType.DMA,
        ],
    )
    def sc_call(emb_hbm, idx_hbm, table_hbm, gam_hbm, bet_hbm, out_hbm,
                idx_v, gam_v, bet_v, emb_v, rows_v, out_v,
                in_s0, in_s1, out_s0, out_s1):
        in_sems = (in_s0, in_s1)
        out_sems = (out_s0, out_s1)
        wid = lax.axis_index("s") * NCORES + lax.axis_index("c")
        base = wid * tok_per_w

        pltpu.sync_copy(idx_hbm.at[wid], idx_v)
        pltpu.sync_copy(gam_hbm, gam_v)
        pltpu.sync_copy(bet_hbm, bet_v)

        def start_in(b, g):
            tok = base + g * T
            pltpu.async_copy(emb_hbm.at[pl.ds(tok, T)], emb_v.at[b], in_sems[b])
            pltpu.async_copy(table_hbm.at[idx_v.at[g]], rows_v.at[b], in_sems[b])

        def wait_in(b, g):
            tok = base + g * T
            pltpu.make_async_copy(emb_hbm.at[pl.ds(tok, T)], emb_v.at[b],
                                  in_sems[b]).wait()
            pltpu.make_async_copy(table_hbm.at[idx_v.at[g]], rows_v.at[b],
                                  in_sems[b]).wait()

        def start_out(b, g):
            tok = base + g * T
            pltpu.async_copy(out_v.at[b], out_hbm.at[pl.ds(tok, T)], out_sems[b])

        def wait_out(b, g):
            tok = base + g * T
            pltpu.make_async_copy(out_v.at[b], out_hbm.at[pl.ds(tok, T)],
                                  out_sems[b]).wait()

        for b in range(NBUF):
            start_in(b, b)

        @pl.loop(0, nchunk, step=NBUF)
        def _chunk(g0):
            for b in range(NBUF):
                g = g0 + b
                wait_in(b, g)

                @pl.when(g >= NBUF)
                def _():
                    wait_out(b, g - NBUF)

                _ln_chunk(emb_v.at[b], rows_v.at[b], out_v.at[b], gam_v, bet_v)
                start_out(b, g)

                @pl.when(g + NBUF < nchunk)
                def _():
                    start_in(b, g + NBUF)

        for b in range(NBUF):
            wait_out(b, nchunk - NBUF + b)

    return sc_call


def kernel(batch_mention_emb, mention_type_ids, table, gamma, beta):
    b, l, h = batch_mention_emb.shape
    n = b * l
    emb = batch_mention_emb.reshape(n, h)
    idx = mention_type_ids.reshape(-1).astype(jnp.int32)
    idx3 = idx.reshape(NW, n // (NW * T), T)
    out = _make_sc_call(n)(emb, idx3, table, gamma, beta)
    return out.reshape(b, l, h)


# speedup vs baseline: 4.1598x; 1.7629x over previous
import functools

import jax
import jax.numpy as jnp
from jax import lax
from jax.experimental import pallas as pl
from jax.experimental.pallas import tpu as pltpu
from jax.experimental.pallas import tpu_sc as plsc

H = 128
LANES = 16
VPR = H // LANES
LP = 64
CB = 2
NBUF = 2
NCORES = 2
NSUB = 16
NW = NCORES * NSUB
EPS = 1e-5


def _allreduce_sum(v):
    lanes = lax.iota(jnp.int32, LANES)
    for sh in (1, 2, 4, 8):
        p = jnp.bitwise_xor(lanes, sh)
        v = v + v.at[p].get(mode="promise_in_bounds", unique_indices=True)
    return v


def _rsqrt(v):
    vi = lax.bitcast_convert_type(v, jnp.int32)
    seed = jnp.full((LANES,), 0x5F3759DF, dtype=jnp.int32) - jnp.right_shift(vi, 1)
    y = lax.bitcast_convert_type(seed, jnp.float32)
    half = v * 0.5
    for _ in range(3):
        y = y * (1.5 - half * y * y)
    return y


def _ln_batch(emb_r, rows_r, out_r, gam_r, bet_r, l_tokens):
    gams = [gam_r[pl.ds(j * LANES, LANES)] for j in range(VPR)]
    bets = [bet_r[pl.ds(j * LANES, LANES)] for j in range(VPR)]

    @pl.loop(0, l_tokens)
    def _token(t):
        x = [emb_r[t, pl.ds(j * LANES, LANES)] + rows_r[t, pl.ds(j * LANES, LANES)]
             for j in range(VPR)]
        s = x[0]
        sq = x[0] * x[0]
        for j in range(1, VPR):
            s = s + x[j]
            sq = sq + x[j] * x[j]
        m_v = _allreduce_sum(s) * (1.0 / H)
        var = _allreduce_sum(sq) * (1.0 / H) - m_v * m_v + EPS
        r_v = _rsqrt(var)
        for j in range(VPR):
            out_r[t, pl.ds(j * LANES, LANES)] = (x[j] - m_v) * r_v * gams[j] + bets[j]


def _make_sc_call(b_total, l_tokens):
    b_per_w = b_total // NW
    nchunk = b_per_w // CB
    mesh = plsc.VectorSubcoreMesh(core_axis_name="c", subcore_axis_name="s",
                                  num_cores=NCORES, num_subcores=NSUB)

    @functools.partial(
        pl.kernel,
        out_type=jax.ShapeDtypeStruct((b_total, l_tokens, H), jnp.float32),
        mesh=mesh,
        compiler_params=pltpu.CompilerParams(use_tc_tiling_on_sc=True),
        scratch_types=[
            pltpu.VMEM((b_per_w * LP,), jnp.int32),
            pltpu.VMEM((H,), jnp.float32),
            pltpu.VMEM((H,), jnp.float32),
            pltpu.VMEM((NBUF, CB, l_tokens, H), jnp.float32),
            pltpu.VMEM((NBUF, CB, l_tokens, H), jnp.float32),
            pltpu.VMEM((NBUF, CB, l_tokens, H), jnp.float32),
            pltpu.SemaphoreType.DMA,
            pltpu.SemaphoreType.DMA,
            pltpu.SemaphoreType.DMA,
            pltpu.SemaphoreType.DMA,
        ],
    )
    def sc_call(emb_hbm, idx_hbm, table_hbm, gam_hbm, bet_hbm, out_hbm,
                idx_v, gam_v, bet_v, emb_v, rows_v, out_v,
                in_s0, in_s1, out_s0, out_s1):
        in_sems = (in_s0, in_s1)
        out_sems = (out_s0, out_s1)
        wid = lax.axis_index("s") * NCORES + lax.axis_index("c")
        base = wid * b_per_w

        pltpu.sync_copy(idx_hbm.at[pl.ds(base * LP, b_per_w * LP)], idx_v)
        pltpu.sync_copy(gam_hbm, gam_v)
        pltpu.sync_copy(bet_hbm, bet_v)

        def start_in(b, g):
            b0 = base + g * CB
            pltpu.async_copy(emb_hbm.at[pl.ds(b0, CB)], emb_v.at[b], in_sems[b])
            for bb in range(CB):
                pltpu.async_copy(
                    table_hbm.at[idx_v.at[pl.ds((g * CB + bb) * LP, l_tokens)]],
                    rows_v.at[b, bb], in_sems[b])

        def wait_in(b, g):
            b0 = base + g * CB
            pltpu.make_async_copy(emb_hbm.at[pl.ds(b0, CB)], emb_v.at[b],
                                  in_sems[b]).wait()
            for bb in range(CB):
                pltpu.make_async_copy(
                    table_hbm.at[idx_v.at[pl.ds((g * CB + bb) * LP, l_tokens)]],
                    rows_v.at[b, bb], in_sems[b]).wait()

        def start_out(b, g):
            b0 = base + g * CB
            pltpu.async_copy(out_v.at[b], out_hbm.at[pl.ds(b0, CB)], out_sems[b])

        def wait_out(b, g):
            b0 = base + g * CB
            pltpu.make_async_copy(out_v.at[b], out_hbm.at[pl.ds(b0, CB)],
                                  out_sems[b]).wait()

        for b in range(NBUF):
            start_in(b, b)

        @pl.loop(0, nchunk, step=NBUF)
        def _chunk(g0):
            for b in range(NBUF):
                g = g0 + b
                wait_in(b, g)

                @pl.when(g >= NBUF)
                def _():
                    wait_out(b, g - NBUF)

                for bb in range(CB):
                    _ln_batch(emb_v.at[b, bb], rows_v.at[b, bb],
                              out_v.at[b, bb], gam_v, bet_v, l_tokens)
                start_out(b, g)

                @pl.when(g + NBUF < nchunk)
                def _():
                    start_in(b, g + NBUF)

        for b in range(NBUF):
            wait_out(b, nchunk - NBUF + b)

    return sc_call


def kernel(batch_mention_emb, mention_type_ids, table, gamma, beta):
    b, l, h = batch_mention_emb.shape
    idx = mention_type_ids.astype(jnp.int32)
    idx_pad = jnp.pad(idx, ((0, 0), (0, LP - l))).reshape(-1)
    return _make_sc_call(b, l)(batch_mention_emb, idx_pad, table, gamma, beta)


# speedup vs baseline: 8.0084x; 1.9252x over previous
import functools

import jax
import jax.numpy as jnp
from jax import lax
from jax.experimental import pallas as pl
from jax.experimental.pallas import tpu as pltpu
from jax.experimental.pallas import tpu_sc as plsc

H = 128
LANES = 16
VPR = H // LANES
T = 64
NBUF = 4
NCORES = 2
NSUB = 16
NW = NCORES * NSUB
EPS = 1e-5


def _allreduce_sum(v, perms):
    for p in perms:
        v = v + v.at[p].get(mode="promise_in_bounds", unique_indices=True)
    return v


def _rsqrt(v):
    vi = lax.bitcast_convert_type(v, jnp.int32)
    seed = jnp.full((LANES,), 0x5F3759DF, dtype=jnp.int32) - jnp.right_shift(vi, 1)
    y = lax.bitcast_convert_type(seed, jnp.float32)
    half = v * 0.5
    for _ in range(2):
        y = y * (1.5 - half * y * y)
    return y


def _ln_chunk(emb_r, rows_r, out_r, gams, bets, perms):
    @pl.loop(0, T)
    def _token(t):
        x = [emb_r[t, pl.ds(j * LANES, LANES)] + rows_r[t, pl.ds(j * LANES, LANES)]
             for j in range(VPR)]
        s = x[0]
        sq = x[0] * x[0]
        for j in range(1, VPR):
            s = s + x[j]
            sq = sq + x[j] * x[j]
        m_v = _allreduce_sum(s, perms) * (1.0 / H)
        var = _allreduce_sum(sq, perms) * (1.0 / H) - m_v * m_v + EPS
        r_v = _rsqrt(var)
        for j in range(VPR):
            out_r[t, pl.ds(j * LANES, LANES)] = (x[j] - m_v) * r_v * gams[j] + bets[j]


def _make_sc_call(n_tokens):
    tok_per_w = n_tokens // NW
    nchunk = tok_per_w // T
    mesh = plsc.VectorSubcoreMesh(core_axis_name="c", subcore_axis_name="s",
                                  num_cores=NCORES, num_subcores=NSUB)

    @functools.partial(
        pl.kernel,
        out_type=jax.ShapeDtypeStruct((n_tokens, H), jnp.float32),
        mesh=mesh,
        scratch_types=[
            pltpu.VMEM((nchunk, T), jnp.int32),
            pltpu.VMEM((H,), jnp.float32),
            pltpu.VMEM((H,), jnp.float32),
            pltpu.VMEM((NBUF, T, H), jnp.float32),
            pltpu.VMEM((NBUF, T, H), jnp.float32),
            pltpu.VMEM((NBUF, T, H), jnp.float32),
        ] + [pltpu.SemaphoreType.DMA] * (2 * NBUF),
    )
    def sc_call(emb_hbm, idx_hbm, table_hbm, gam_hbm, bet_hbm, out_hbm,
                idx_v, gam_v, bet_v, emb_v, rows_v, out_v, *sems):
        in_sems = sems[:NBUF]
        out_sems = sems[NBUF:]
        wid = lax.axis_index("s") * NCORES + lax.axis_index("c")
        base = wid * tok_per_w

        pltpu.sync_copy(idx_hbm.at[wid], idx_v)
        pltpu.sync_copy(gam_hbm, gam_v)
        pltpu.sync_copy(bet_hbm, bet_v)

        lanes = lax.iota(jnp.int32, LANES)
        perms = [jnp.bitwise_xor(lanes, sh) for sh in (1, 2, 4, 8)]
        gams = [gam_v[pl.ds(j * LANES, LANES)] for j in range(VPR)]
        bets = [bet_v[pl.ds(j * LANES, LANES)] for j in range(VPR)]

        def start_in(b, g):
            tok = base + g * T
            pltpu.async_copy(emb_hbm.at[pl.ds(tok, T)], emb_v.at[b], in_sems[b])
            pltpu.async_copy(table_hbm.at[idx_v.at[g]], rows_v.at[b], in_sems[b])

        def wait_in(b, g):
            tok = base + g * T
            pltpu.make_async_copy(emb_hbm.at[pl.ds(tok, T)], emb_v.at[b],
                                  in_sems[b]).wait()
            pltpu.make_async_copy(table_hbm.at[idx_v.at[g]], rows_v.at[b],
                                  in_sems[b]).wait()

        def start_out(b, g):
            tok = base + g * T
            pltpu.async_copy(out_v.at[b], out_hbm.at[pl.ds(tok, T)], out_sems[b])

        def wait_out(b, g):
            tok = base + g * T
            pltpu.make_async_copy(out_v.at[b], out_hbm.at[pl.ds(tok, T)],
                                  out_sems[b]).wait()

        for b in range(NBUF):
            start_in(b, b)

        @pl.loop(0, nchunk, step=NBUF)
        def _chunk(g0):
            for b in range(NBUF):
                g = g0 + b
                wait_in(b, g)

                @pl.when(g >= NBUF)
                def _():
                    wait_out(b, g - NBUF)

                _ln_chunk(emb_v.at[b], rows_v.at[b], out_v.at[b],
                          gams, bets, perms)
                start_out(b, g)

                @pl.when(g + NBUF < nchunk)
                def _():
                    start_in(b, g + NBUF)

        for b in range(NBUF):
            wait_out(b, nchunk - NBUF + b)

    return sc_call


def kernel(batch_mention_emb, mention_type_ids, table, gamma, beta):
    b, l, h = batch_mention_emb.shape
    n = b * l
    emb = jnp.transpose(batch_mention_emb, (1, 0, 2)).reshape(n, h)
    idx = jnp.transpose(mention_type_ids).astype(jnp.int32).reshape(-1)
    idx3 = idx.reshape(NW, n // (NW * T), T)
    out = _make_sc_call(n)(emb, idx3, table, gamma, beta)
    return jnp.transpose(out.reshape(l, b, h), (1, 0, 2))
